# baseline (device time: 19768 ns/iter reference)
import jax
import jax.numpy as jnp
from jax import lax
from jax.experimental import pallas as pl
from jax.experimental.pallas import tpu as pltpu

N_DEV = 4
N_TOK = 256
D_IN = 128
D_OUT = 256
N_EXP = 8
EXP_PER_DEV = N_EXP // N_DEV
CAPACITY = 25


def kernel(x, router_W, route_idx, expert_W):
    del router_W

    def body(x_ref, idx_ref, ew_ref, out_ref, comm_ref, send_sems, recv_sems):
        my_pos = lax.axis_index("i")
        left = lax.rem(my_pos + N_DEV - 1, N_DEV)
        right = lax.rem(my_pos + 1, N_DEV)

        barrier_sem = pltpu.get_barrier_semaphore()
        for nbr in (left, right):
            pl.semaphore_signal(
                barrier_sem, inc=1,
                device_id=(nbr,), device_id_type=pl.DeviceIdType.MESH,
            )
        pl.semaphore_wait(barrier_sem, 2)

        idx = idx_ref[:, :]
        e_ids = lax.broadcasted_iota(jnp.int32, (N_TOK, N_EXP), 1)
        onehot = (idx == e_ids).astype(jnp.float32)

        row = lax.broadcasted_iota(jnp.int32, (N_TOK, N_TOK), 0)
        col = lax.broadcasted_iota(jnp.int32, (N_TOK, N_TOK), 1)
        tri = (row >= col).astype(jnp.float32)
        pos = jnp.dot(tri, onehot, preferred_element_type=jnp.float32)
        keep = onehot * (pos <= CAPACITY).astype(jnp.float32)

        xv = x_ref[:, :]
        acc = jnp.zeros((N_TOK, D_OUT), jnp.float32)
        for le in range(EXP_PER_DEV):
            ge = EXP_PER_DEV * my_pos + le
            sel = (e_ids == ge).astype(jnp.float32)
            tok = jnp.sum(keep * sel, axis=1, keepdims=True)
            acc = acc + jnp.dot(
                xv * tok, ew_ref[le], preferred_element_type=jnp.float32
            )

        out_ref[:, :] = acc
        comm_ref[0, :, :] = acc

        for h in range(N_DEV - 1):
            send_slot = h % 2
            recv_slot = (h + 1) % 2
            rdma = pltpu.make_async_remote_copy(
                src_ref=comm_ref.at[send_slot],
                dst_ref=comm_ref.at[recv_slot],
                send_sem=send_sems.at[h],
                recv_sem=recv_sems.at[h],
                device_id=(right,),
                device_id_type=pl.DeviceIdType.MESH,
            )
            rdma.start()
            rdma.wait()
            out_ref[:, :] += comm_ref[recv_slot, :, :]

    return pl.pallas_call(
        body,
        out_shape=jax.ShapeDtypeStruct((N_TOK, D_OUT), jnp.float32),
        in_specs=[
            pl.BlockSpec(memory_space=pltpu.VMEM),
            pl.BlockSpec(memory_space=pltpu.VMEM),
            pl.BlockSpec(memory_space=pltpu.VMEM),
        ],
        out_specs=pl.BlockSpec(memory_space=pltpu.VMEM),
        scratch_shapes=[
            pltpu.VMEM((2, N_TOK, D_OUT), jnp.float32),
            pltpu.SemaphoreType.DMA((N_DEV - 1,)),
            pltpu.SemaphoreType.DMA((N_DEV - 1,)),
        ],
        compiler_params=pltpu.CompilerParams(collective_id=0),
    )(x, route_idx, expert_W)


# device time: 14864 ns/iter; 1.3299x vs baseline; 1.3299x over previous
import jax
import jax.numpy as jnp
from jax import lax
from jax.experimental import pallas as pl
from jax.experimental.pallas import tpu as pltpu

N_DEV = 4
N_TOK = 256
D_IN = 128
D_OUT = 256
N_EXP = 8
EXP_PER_DEV = N_EXP // N_DEV
CAPACITY = 25


def kernel(x, router_W, route_idx, expert_W):
    del router_W

    def body(x_ref, idx_ref, ew_ref, out_ref, comm_ref, send_sems, recv_sems):
        my_pos = lax.axis_index("i")
        left = lax.rem(my_pos + N_DEV - 1, N_DEV)
        right = lax.rem(my_pos + 1, N_DEV)

        barrier_sem = pltpu.get_barrier_semaphore()
        for nbr in (left, right):
            pl.semaphore_signal(
                barrier_sem, inc=1,
                device_id=(nbr,), device_id_type=pl.DeviceIdType.MESH,
            )
        pl.semaphore_wait(barrier_sem, 2)

        idx = idx_ref[:, :]
        e_ids = lax.broadcasted_iota(jnp.int32, (N_TOK, N_EXP), 1)
        onehot = (idx == e_ids).astype(jnp.float32)

        row = lax.broadcasted_iota(jnp.int32, (N_TOK, N_TOK), 0)
        col = lax.broadcasted_iota(jnp.int32, (N_TOK, N_TOK), 1)
        tri = (row >= col).astype(jnp.float32)
        pos = jnp.dot(tri, onehot, preferred_element_type=jnp.float32)
        keep = onehot * (pos <= CAPACITY).astype(jnp.float32)

        xv = x_ref[:, :]
        acc = jnp.zeros((N_TOK, D_OUT), jnp.float32)
        for le in range(EXP_PER_DEV):
            ge = EXP_PER_DEV * my_pos + le
            sel = (e_ids == ge).astype(jnp.float32)
            tok = jnp.sum(keep * sel, axis=1, keepdims=True)
            acc = acc + jnp.dot(
                xv * tok, ew_ref[le], preferred_element_type=jnp.float32
            )

        out_ref[:, :] = acc

        is_even = lax.rem(my_pos, 2) == 0
        partner1 = jnp.where(is_even, right, left)
        partner2 = jnp.where(is_even, left, right)
        for r, partner in enumerate((partner1, partner2)):
            rdma = pltpu.make_async_remote_copy(
                src_ref=out_ref,
                dst_ref=comm_ref.at[r],
                send_sem=send_sems.at[r],
                recv_sem=recv_sems.at[r],
                device_id=(partner,),
                device_id_type=pl.DeviceIdType.MESH,
            )
            rdma.start()
            rdma.wait()
            out_ref[:, :] += comm_ref[r, :, :]

    return pl.pallas_call(
        body,
        out_shape=jax.ShapeDtypeStruct((N_TOK, D_OUT), jnp.float32),
        in_specs=[
            pl.BlockSpec(memory_space=pltpu.VMEM),
            pl.BlockSpec(memory_space=pltpu.VMEM),
            pl.BlockSpec(memory_space=pltpu.VMEM),
        ],
        out_specs=pl.BlockSpec(memory_space=pltpu.VMEM),
        scratch_shapes=[
            pltpu.VMEM((2, N_TOK, D_OUT), jnp.float32),
            pltpu.SemaphoreType.DMA((2,)),
            pltpu.SemaphoreType.DMA((2,)),
        ],
        compiler_params=pltpu.CompilerParams(collective_id=0),
    )(x, route_idx, expert_W)


# device time: 13445 ns/iter; 1.4703x vs baseline; 1.1055x over previous
import jax
import jax.numpy as jnp
from jax import lax
from jax.experimental import pallas as pl
from jax.experimental.pallas import tpu as pltpu

N_DEV = 4
N_TOK = 256
D_IN = 128
D_OUT = 256
N_EXP = 8
EXP_PER_DEV = N_EXP // N_DEV
CAPACITY = 25


def kernel(x, router_W, route_idx, expert_W):
    del router_W

    def body(x_ref, idx_ref, ew_ref, out_ref, comm_ref, send_sems, recv_sems):
        my_pos = lax.axis_index("i")
        left = lax.rem(my_pos + N_DEV - 1, N_DEV)
        right = lax.rem(my_pos + 1, N_DEV)

        barrier_sem = pltpu.get_barrier_semaphore()
        for nbr in (left, right):
            pl.semaphore_signal(
                barrier_sem, inc=1,
                device_id=(nbr,), device_id_type=pl.DeviceIdType.MESH,
            )
        pl.semaphore_wait(barrier_sem, 2)

        idx = idx_ref[:, :]
        e_ids = lax.broadcasted_iota(jnp.int32, (N_TOK, N_EXP), 1)
        onehot = (idx == e_ids).astype(jnp.float32)

        row = lax.broadcasted_iota(jnp.int32, (N_TOK, N_TOK), 0)
        col = lax.broadcasted_iota(jnp.int32, (N_TOK, N_TOK), 1)
        tri = (row >= col).astype(jnp.float32)
        pos = jnp.dot(tri, onehot, preferred_element_type=jnp.float32)
        keep = onehot * (pos <= CAPACITY).astype(jnp.float32)

        xv = x_ref[:, :]
        xm = []
        for le in range(EXP_PER_DEV):
            ge = EXP_PER_DEV * my_pos + le
            sel = (e_ids == ge).astype(jnp.float32)
            tok = jnp.sum(keep * sel, axis=1, keepdims=True)
            xm.append(xv * tok)

        is_even = lax.rem(my_pos, 2) == 0
        partner1 = jnp.where(is_even, right, left)
        partner2 = jnp.where(is_even, left, right)
        H = D_OUT // 2

        def exchange(r, h, partner):
            k = 2 * r + h
            return pltpu.make_async_remote_copy(
                src_ref=out_ref.at[:, pl.ds(h * H, H)],
                dst_ref=comm_ref.at[k],
                send_sem=send_sems.at[k],
                recv_sem=recv_sems.at[k],
                device_id=(partner,),
                device_id_type=pl.DeviceIdType.MESH,
            )

        for h in range(2):
            cols = pl.ds(h * H, H)
            out_ref[:, cols] = jnp.dot(
                xm[0], ew_ref[0, :, cols], preferred_element_type=jnp.float32
            ) + jnp.dot(
                xm[1], ew_ref[1, :, cols], preferred_element_type=jnp.float32
            )
            r1 = exchange(0, h, partner1)
            r1.start()
            if h == 0:
                r1a = r1
        r1b = r1

        r2 = [None, None]
        for h, rdma in enumerate((r1a, r1b)):
            rdma.wait()
            cols = pl.ds(h * H, H)
            out_ref[:, cols] += comm_ref[2 * 0 + h, :, :]
            r2[h] = exchange(1, h, partner2)
            r2[h].start()
        for h in range(2):
            r2[h].wait()
            cols = pl.ds(h * H, H)
            out_ref[:, cols] += comm_ref[2 * 1 + h, :, :]

    return pl.pallas_call(
        body,
        out_shape=jax.ShapeDtypeStruct((N_TOK, D_OUT), jnp.float32),
        in_specs=[
            pl.BlockSpec(memory_space=pltpu.VMEM),
            pl.BlockSpec(memory_space=pltpu.VMEM),
            pl.BlockSpec(memory_space=pltpu.VMEM),
        ],
        out_specs=pl.BlockSpec(memory_space=pltpu.VMEM),
        scratch_shapes=[
            pltpu.VMEM((4, N_TOK, D_OUT // 2), jnp.float32),
            pltpu.SemaphoreType.DMA((4,)),
            pltpu.SemaphoreType.DMA((4,)),
        ],
        compiler_params=pltpu.CompilerParams(collective_id=0),
    )(x, route_idx, expert_W)
